# Initial kernel scaffold; baseline (speedup 1.0000x reference)
import functools

import jax
import jax.numpy as jnp
from jax import lax
from jax.experimental import pallas as pl
from jax.experimental.pallas import tpu as pltpu
from jax.experimental.pallas import tpu_sc as plsc

N = 10000
E = 320000
D = 128
H = 4
HP = 16

NC = 2
NS = 16
NW = NC * NS
EB = 128
RPS = N // NS

_R1 = 1000
_G1 = N // _R1

_mesh = plsc.VectorSubcoreMesh(core_axis_name="c", subcore_axis_name="s")


def _k1_body(x_ref, wt_ref, attl_ref, attr_ref, g_ref, b_ref,
             xl_ref, al_ref, ar_ref, cal_ref, car_ref):
    i = pl.program_id(0)
    x = x_ref[...]
    mu = jnp.mean(x, axis=-1, keepdims=True)
    xc = x - mu
    var = jnp.mean(xc * xc, axis=-1, keepdims=True)
    h = g_ref[...] * xc * lax.rsqrt(var + 1e-5) + b_ref[...]
    xl = jnp.dot(h, wt_ref[...], preferred_element_type=jnp.float32)
    xl_ref[...] = xl
    al = jnp.dot(xl, attl_ref[...], preferred_element_type=jnp.float32)
    ar = jnp.dot(xl, attr_ref[...], preferred_element_type=jnp.float32)
    al_ref[...] = al
    ar_ref[...] = ar
    bl = jnp.max(al, axis=0, keepdims=True)
    br = jnp.max(ar, axis=0, keepdims=True)

    @pl.when(i == 0)
    def _():
        cal_ref[...] = bl
        car_ref[...] = br

    @pl.when(i > 0)
    def _():
        cal_ref[...] = jnp.maximum(cal_ref[...], bl)
        car_ref[...] = jnp.maximum(car_ref[...], br)


_k1 = pl.pallas_call(
    _k1_body,
    grid=(_G1,),
    in_specs=[
        pl.BlockSpec((_R1, D), lambda i: (i, 0)),
        pl.BlockSpec((D, D), lambda i: (0, 0)),
        pl.BlockSpec((D, HP), lambda i: (0, 0)),
        pl.BlockSpec((D, HP), lambda i: (0, 0)),
        pl.BlockSpec((1, D), lambda i: (0, 0)),
        pl.BlockSpec((1, D), lambda i: (0, 0)),
    ],
    out_specs=[
        pl.BlockSpec((_R1, D), lambda i: (i, 0)),
        pl.BlockSpec((_R1, HP), lambda i: (i, 0)),
        pl.BlockSpec((_R1, HP), lambda i: (i, 0)),
        pl.BlockSpec((1, HP), lambda i: (0, 0)),
        pl.BlockSpec((1, HP), lambda i: (0, 0)),
    ],
    out_shape=[
        jax.ShapeDtypeStruct((N, D), jnp.float32),
        jax.ShapeDtypeStruct((N, HP), jnp.float32),
        jax.ShapeDtypeStruct((N, HP), jnp.float32),
        jax.ShapeDtypeStruct((1, HP), jnp.float32),
        jax.ShapeDtypeStruct((1, HP), jnp.float32),
    ],
)


@functools.partial(
    pl.kernel,
    out_type=[
        jax.ShapeDtypeStruct((E, HP), jnp.float32),
        jax.ShapeDtypeStruct((NC, N, HP), jnp.float32),
    ],
    mesh=_mesh,
    scratch_types=[
        pltpu.VMEM((EB,), jnp.int32),
        pltpu.VMEM((EB,), jnp.int32),
        pltpu.VMEM((EB, HP), jnp.float32),
        pltpu.VMEM((EB, HP), jnp.float32),
        pltpu.VMEM((16,), jnp.float32),
        pltpu.VMEM((16,), jnp.float32),
        pltpu.VMEM_SHARED((N, HP), jnp.float32),
    ],
)
def _k2(adj_hbm, al_hbm, ar_hbm, cal_hbm, car_hbm, z16_hbm,
        ex_hbm, dpart_hbm,
        dsti, srci, gal, gar, calv, carv, dshared):
    cid = lax.axis_index("c")
    sid = lax.axis_index("s")
    wid = sid * NC + cid
    pltpu.sync_copy(z16_hbm.at[pl.ds(sid * RPS, RPS)],
                    dshared.at[pl.ds(sid * RPS, RPS)])
    pltpu.sync_copy(cal_hbm.at[0], calv)
    pltpu.sync_copy(car_hbm.at[0], carv)
    plsc.subcore_barrier()
    c = jnp.maximum(calv[...] + carv[...], 0.0)

    @pl.loop(wid * EB, E, step=NW * EB)
    def _(e0):
        pltpu.sync_copy(adj_hbm.at[1, pl.ds(e0, EB)], dsti)
        pltpu.sync_copy(adj_hbm.at[0, pl.ds(e0, EB)], srci)
        pltpu.sync_copy(al_hbm.at[dsti], gal)
        pltpu.sync_copy(ar_hbm.at[srci], gar)

        @pl.loop(0, EB)
        def _(j):
            s = gal[j, :] + gar[j, :]
            s = jnp.maximum(s, 0.2 * s)
            gal[j, :] = jnp.exp(s - c)

        pltpu.sync_copy(gal, ex_hbm.at[pl.ds(e0, EB)])
        pltpu.sync_copy(gal, dshared.at[dsti], add=True)

    plsc.subcore_barrier()
    pltpu.sync_copy(dshared.at[pl.ds(sid * RPS, RPS)],
                    dpart_hbm.at[cid, pl.ds(sid * RPS, RPS)])


def _k3_body(dp_ref, rcp_ref):
    d = dp_ref[0] + dp_ref[1]
    lanes = lax.broadcasted_iota(jnp.int32, (1, HP), 1)
    rcp_ref[...] = jnp.where(lanes < H, 1.0 / (d + 1e-16), 0.0)


_k3 = pl.pallas_call(
    _k3_body,
    out_shape=jax.ShapeDtypeStruct((N, HP), jnp.float32),
)


@functools.partial(
    pl.kernel,
    out_type=jax.ShapeDtypeStruct((NC, N, D), jnp.float32),
    mesh=_mesh,
    scratch_types=[
        pltpu.VMEM((EB,), jnp.int32),
        pltpu.VMEM((EB,), jnp.int32),
        pltpu.VMEM((EB, HP), jnp.float32),
        pltpu.VMEM((EB, HP), jnp.float32),
        pltpu.VMEM((EB, D), jnp.float32),
        pltpu.VMEM_SHARED((N, D), jnp.float32),
    ],
)
def _k4(adj_hbm, xl_hbm, ex_hbm, rcp_hbm, znd_hbm,
        opart_hbm,
        dsti, srci, exb, rcpb, rows, oshared):
    cid = lax.axis_index("c")
    sid = lax.axis_index("s")
    wid = sid * NC + cid
    pltpu.sync_copy(znd_hbm.at[pl.ds(sid * RPS, RPS)],
                    oshared.at[pl.ds(sid * RPS, RPS)])
    plsc.subcore_barrier()

    @pl.loop(wid * EB, E, step=NW * EB)
    def _(e0):
        pltpu.sync_copy(adj_hbm.at[1, pl.ds(e0, EB)], dsti)
        pltpu.sync_copy(adj_hbm.at[0, pl.ds(e0, EB)], srci)
        pltpu.sync_copy(ex_hbm.at[pl.ds(e0, EB)], exb)
        pltpu.sync_copy(rcp_hbm.at[dsti], rcpb)
        pltpu.sync_copy(xl_hbm.at[srci], rows)

        @pl.loop(0, EB)
        def _(j):
            p = exb[j, :] * rcpb[j, :]
            w = 0.25 * jnp.sum(p)

            @pl.loop(0, D, step=16)
            def _(k):
                rows[j, pl.ds(k, 16)] = rows[j, pl.ds(k, 16)] * w

        pltpu.sync_copy(rows, oshared.at[dsti], add=True)

    plsc.subcore_barrier()
    pltpu.sync_copy(oshared.at[pl.ds(sid * RPS, RPS)],
                    opart_hbm.at[cid, pl.ds(sid * RPS, RPS)])


def _k5_body(x_ref, o0_ref, o1_ref, b_ref, out_ref):
    out_ref[...] = x_ref[...] + o0_ref[0] + o1_ref[0] + b_ref[...]


_k5 = pl.pallas_call(
    _k5_body,
    grid=(_G1,),
    in_specs=[
        pl.BlockSpec((_R1, D), lambda i: (i, 0)),
        pl.BlockSpec((1, _R1, D), lambda i: (0, i, 0)),
        pl.BlockSpec((1, _R1, D), lambda i: (1, i, 0)),
        pl.BlockSpec((1, D), lambda i: (0, 0)),
    ],
    out_specs=pl.BlockSpec((_R1, D), lambda i: (i, 0)),
    out_shape=jax.ShapeDtypeStruct((N, D), jnp.float32),
)


def kernel(x, adj, W, att_l, att_r, bias, gamma, beta):
    Wt = W.T
    attl16 = jnp.pad(att_l, ((0, 0), (0, HP - H)))
    attr16 = jnp.pad(att_r, ((0, 0), (0, HP - H)))
    xl, al16, ar16, cal, car = _k1(x, Wt, attl16, attr16,
                                   gamma.reshape(1, D), beta.reshape(1, D))
    z16 = jnp.zeros((N, HP), jnp.float32)
    ex, dpart = _k2(adj, al16, ar16, cal, car, z16)
    rcp = _k3(dpart)
    znd = jnp.zeros((N, D), jnp.float32)
    opart = _k4(adj, xl, ex, rcp, znd)
    return _k5(x, opart, opart, bias.reshape(1, D))


# speedup vs baseline: 7.5491x; 7.5491x over previous
import functools

import jax
import jax.numpy as jnp
from jax import lax
from jax.experimental import pallas as pl
from jax.experimental.pallas import tpu as pltpu
from jax.experimental.pallas import tpu_sc as plsc

N = 10000
E = 320000
D = 128
H = 4
HP = 16

NC = 2
NS = 16
NW = NC * NS
EB = 128
NP = 10240
RPS = NP // NS

_R1 = 1000
_G1 = N // _R1

_mesh = plsc.VectorSubcoreMesh(core_axis_name="c", subcore_axis_name="s")
_sc_params = pltpu.CompilerParams(use_tc_tiling_on_sc=False,
                                  needs_layout_passes=False)


def _k1_body(x_ref, wt_ref, attl_ref, attr_ref, g_ref, b_ref,
             xl_ref, al_ref, ar_ref, cal_ref, car_ref):
    i = pl.program_id(0)
    x = x_ref[...]
    mu = jnp.mean(x, axis=-1, keepdims=True)
    xc = x - mu
    var = jnp.mean(xc * xc, axis=-1, keepdims=True)
    h = g_ref[...] * xc * lax.rsqrt(var + 1e-5) + b_ref[...]
    xl = jnp.dot(h, wt_ref[...], preferred_element_type=jnp.float32)
    xl_ref[...] = xl
    al = jnp.dot(xl, attl_ref[...], preferred_element_type=jnp.float32)
    ar = jnp.dot(xl, attr_ref[...], preferred_element_type=jnp.float32)
    al_ref[...] = al
    ar_ref[...] = ar
    bl = jnp.max(al, axis=0, keepdims=True)
    br = jnp.max(ar, axis=0, keepdims=True)

    @pl.when(i == 0)
    def _():
        cal_ref[...] = bl
        car_ref[...] = br

    @pl.when(i > 0)
    def _():
        cal_ref[...] = jnp.maximum(cal_ref[...], bl)
        car_ref[...] = jnp.maximum(car_ref[...], br)


_k1 = pl.pallas_call(
    _k1_body,
    grid=(_G1,),
    in_specs=[
        pl.BlockSpec((_R1, D), lambda i: (i, 0)),
        pl.BlockSpec((D, D), lambda i: (0, 0)),
        pl.BlockSpec((D, HP), lambda i: (0, 0)),
        pl.BlockSpec((D, HP), lambda i: (0, 0)),
        pl.BlockSpec((1, D), lambda i: (0, 0)),
        pl.BlockSpec((1, D), lambda i: (0, 0)),
    ],
    out_specs=[
        pl.BlockSpec((_R1, D), lambda i: (i, 0)),
        pl.BlockSpec((_R1, HP), lambda i: (i, 0)),
        pl.BlockSpec((_R1, HP), lambda i: (i, 0)),
        pl.BlockSpec((1, HP), lambda i: (0, 0)),
        pl.BlockSpec((1, HP), lambda i: (0, 0)),
    ],
    out_shape=[
        jax.ShapeDtypeStruct((N, D), jnp.float32),
        jax.ShapeDtypeStruct((N, HP), jnp.float32),
        jax.ShapeDtypeStruct((N, HP), jnp.float32),
        jax.ShapeDtypeStruct((1, HP), jnp.float32),
        jax.ShapeDtypeStruct((1, HP), jnp.float32),
    ],
)


@functools.partial(
    pl.kernel,
    out_type=[
        jax.ShapeDtypeStruct((E, HP), jnp.float32),
        jax.ShapeDtypeStruct((NC, NP, HP), jnp.float32),
    ],
    mesh=_mesh,
    scratch_types=[
        pltpu.VMEM((EB,), jnp.int32),
        pltpu.VMEM((EB,), jnp.int32),
        pltpu.VMEM((EB, HP), jnp.float32),
        pltpu.VMEM((EB, HP), jnp.float32),
        pltpu.VMEM((16,), jnp.float32),
        pltpu.VMEM((16,), jnp.float32),
        pltpu.VMEM_SHARED((NP, HP), jnp.float32),
    ],
    compiler_params=_sc_params,
)
def _k2(dst_hbm, src_hbm, al_hbm, ar_hbm, cal_hbm, car_hbm, z16_hbm,
        ex_hbm, dpart_hbm,
        dsti, srci, gal, gar, calv, carv, dshared):
    cid = lax.axis_index("c")
    sid = lax.axis_index("s")
    wid = sid * NC + cid
    pltpu.sync_copy(z16_hbm.at[pl.ds(sid * RPS, RPS)],
                    dshared.at[pl.ds(sid * RPS, RPS)])
    pltpu.sync_copy(cal_hbm.at[0], calv)
    pltpu.sync_copy(car_hbm.at[0], carv)
    plsc.subcore_barrier()
    c = jnp.maximum(calv[...] + carv[...], 0.0)

    @pl.loop(wid * EB, E, step=NW * EB)
    def _(e0):
        pltpu.sync_copy(dst_hbm.at[pl.ds(e0, EB)], dsti)
        pltpu.sync_copy(src_hbm.at[pl.ds(e0, EB)], srci)
        pltpu.sync_copy(al_hbm.at[dsti], gal)
        pltpu.sync_copy(ar_hbm.at[srci], gar)

        @pl.loop(0, EB)
        def _(j):
            s = gal[j, :] + gar[j, :]
            s = jnp.maximum(s, 0.2 * s)
            gal[j, :] = jnp.exp(s - c)

        pltpu.sync_copy(gal, ex_hbm.at[pl.ds(e0, EB)])
        pltpu.sync_copy(gal, dshared.at[dsti], add=True)

    plsc.subcore_barrier()
    pltpu.sync_copy(dshared.at[pl.ds(sid * RPS, RPS)],
                    dpart_hbm.at[cid, pl.ds(sid * RPS, RPS)])


def _k3_body(dp_ref, rcp_ref):
    d = dp_ref[0] + dp_ref[1]
    lanes = lax.broadcasted_iota(jnp.int32, (1, HP), 1)
    rcp_ref[...] = jnp.where(lanes < H, 1.0 / (d + 1e-16), 0.0)


_k3 = pl.pallas_call(
    _k3_body,
    out_shape=jax.ShapeDtypeStruct((NP, HP), jnp.float32),
)


@functools.partial(
    pl.kernel,
    out_type=jax.ShapeDtypeStruct((NC, NP, D), jnp.float32),
    mesh=_mesh,
    scratch_types=[
        pltpu.VMEM((EB,), jnp.int32),
        pltpu.VMEM((EB,), jnp.int32),
        pltpu.VMEM((EB, HP), jnp.float32),
        pltpu.VMEM((EB, HP), jnp.float32),
        pltpu.VMEM((EB, D), jnp.float32),
        pltpu.VMEM_SHARED((NP, D), jnp.float32),
    ],
    compiler_params=_sc_params,
)
def _k4(dst_hbm, src_hbm, xl_hbm, ex_hbm, rcp_hbm, znd_hbm,
        opart_hbm,
        dsti, srci, exb, rcpb, rows, oshared):
    cid = lax.axis_index("c")
    sid = lax.axis_index("s")
    wid = sid * NC + cid
    pltpu.sync_copy(znd_hbm.at[pl.ds(sid * RPS, RPS)],
                    oshared.at[pl.ds(sid * RPS, RPS)])
    plsc.subcore_barrier()

    @pl.loop(wid * EB, E, step=NW * EB)
    def _(e0):
        pltpu.sync_copy(dst_hbm.at[pl.ds(e0, EB)], dsti)
        pltpu.sync_copy(src_hbm.at[pl.ds(e0, EB)], srci)
        pltpu.sync_copy(ex_hbm.at[pl.ds(e0, EB)], exb)
        pltpu.sync_copy(rcp_hbm.at[dsti], rcpb)
        pltpu.sync_copy(xl_hbm.at[srci], rows)

        @pl.loop(0, EB)
        def _(j):
            p = exb[j, :] * rcpb[j, :]
            w = 0.25 * jnp.sum(p)

            @pl.loop(0, D, step=16)
            def _(k):
                rows[j, pl.ds(k, 16)] = rows[j, pl.ds(k, 16)] * w

        pltpu.sync_copy(rows, oshared.at[dsti], add=True)

    plsc.subcore_barrier()
    pltpu.sync_copy(oshared.at[pl.ds(sid * RPS, RPS)],
                    opart_hbm.at[cid, pl.ds(sid * RPS, RPS)])


def _k5_body(x_ref, o0_ref, o1_ref, b_ref, out_ref):
    out_ref[...] = x_ref[...] + o0_ref[0] + o1_ref[0] + b_ref[...]


_k5 = pl.pallas_call(
    _k5_body,
    grid=(_G1,),
    in_specs=[
        pl.BlockSpec((_R1, D), lambda i: (i, 0)),
        pl.BlockSpec((1, _R1, D), lambda i: (0, i, 0)),
        pl.BlockSpec((1, _R1, D), lambda i: (1, i, 0)),
        pl.BlockSpec((1, D), lambda i: (0, 0)),
    ],
    out_specs=pl.BlockSpec((_R1, D), lambda i: (i, 0)),
    out_shape=jax.ShapeDtypeStruct((N, D), jnp.float32),
)


def kernel(x, adj, W, att_l, att_r, bias, gamma, beta):
    Wt = W.T
    attl16 = jnp.pad(att_l, ((0, 0), (0, HP - H)))
    attr16 = jnp.pad(att_r, ((0, 0), (0, HP - H)))
    xl, al16, ar16, cal, car = _k1(x, Wt, attl16, attr16,
                                   gamma.reshape(1, D), beta.reshape(1, D))
    dst = adj[1]
    src = adj[0]
    z16 = jnp.zeros((NP, HP), jnp.float32)
    ex, dpart = _k2(dst, src, al16, ar16, cal, car, z16)
    rcp = _k3(dpart)
    znd = jnp.zeros((NP, D), jnp.float32)
    opart = _k4(dst, src, xl, ex, rcp, znd)
    return _k5(x, opart, opart, bias.reshape(1, D))


# speedup vs baseline: 13.5673x; 1.7972x over previous
import functools

import jax
import jax.numpy as jnp
from jax import lax
from jax.experimental import pallas as pl
from jax.experimental.pallas import tpu as pltpu
from jax.experimental.pallas import tpu_sc as plsc

N = 10000
E = 320000
D = 128
H = 4
HP = 16

NC = 2
NS = 16
NW = NC * NS
EB = 128
NP = 10240
RPS = NP // NS

_R1 = 1000
_G1 = N // _R1

_mesh = plsc.VectorSubcoreMesh(core_axis_name="c", subcore_axis_name="s")
_sc_params = pltpu.CompilerParams(use_tc_tiling_on_sc=False,
                                  needs_layout_passes=False)


def _k1_body(x_ref, wt_ref, attl_ref, attr_ref, g_ref, b_ref,
             xl_ref, al_ref, ar_ref, cal_ref, car_ref):
    i = pl.program_id(0)
    x = x_ref[...]
    mu = jnp.mean(x, axis=-1, keepdims=True)
    xc = x - mu
    var = jnp.mean(xc * xc, axis=-1, keepdims=True)
    h = g_ref[...] * xc * lax.rsqrt(var + 1e-5) + b_ref[...]
    xl = jnp.dot(h, wt_ref[...], preferred_element_type=jnp.float32)
    xl_ref[...] = xl
    al = jnp.dot(xl, attl_ref[...], preferred_element_type=jnp.float32)
    ar = jnp.dot(xl, attr_ref[...], preferred_element_type=jnp.float32)
    al_ref[...] = al
    ar_ref[...] = ar
    bl = jnp.max(al, axis=0, keepdims=True)
    br = jnp.max(ar, axis=0, keepdims=True)

    @pl.when(i == 0)
    def _():
        cal_ref[...] = bl
        car_ref[...] = br

    @pl.when(i > 0)
    def _():
        cal_ref[...] = jnp.maximum(cal_ref[...], bl)
        car_ref[...] = jnp.maximum(car_ref[...], br)


_k1 = pl.pallas_call(
    _k1_body,
    grid=(_G1,),
    in_specs=[
        pl.BlockSpec((_R1, D), lambda i: (i, 0)),
        pl.BlockSpec((D, D), lambda i: (0, 0)),
        pl.BlockSpec((D, HP), lambda i: (0, 0)),
        pl.BlockSpec((D, HP), lambda i: (0, 0)),
        pl.BlockSpec((1, D), lambda i: (0, 0)),
        pl.BlockSpec((1, D), lambda i: (0, 0)),
    ],
    out_specs=[
        pl.BlockSpec((_R1, D), lambda i: (i, 0)),
        pl.BlockSpec((_R1, HP), lambda i: (i, 0)),
        pl.BlockSpec((_R1, HP), lambda i: (i, 0)),
        pl.BlockSpec((1, HP), lambda i: (0, 0)),
        pl.BlockSpec((1, HP), lambda i: (0, 0)),
    ],
    out_shape=[
        jax.ShapeDtypeStruct((N, D), jnp.float32),
        jax.ShapeDtypeStruct((N, HP), jnp.float32),
        jax.ShapeDtypeStruct((N, HP), jnp.float32),
        jax.ShapeDtypeStruct((1, HP), jnp.float32),
        jax.ShapeDtypeStruct((1, HP), jnp.float32),
    ],
)


_GMAX = 80


@functools.partial(
    pl.kernel,
    out_type=[
        jax.ShapeDtypeStruct((E, HP), jnp.float32),
        jax.ShapeDtypeStruct((NC, NP, HP), jnp.float32),
    ],
    mesh=_mesh,
    scratch_types=[
        pltpu.VMEM((EB,), jnp.int32), pltpu.VMEM((EB,), jnp.int32),
        pltpu.VMEM((EB,), jnp.int32), pltpu.VMEM((EB,), jnp.int32),
        pltpu.VMEM((EB, HP), jnp.float32), pltpu.VMEM((EB, HP), jnp.float32),
        pltpu.VMEM((EB, HP), jnp.float32), pltpu.VMEM((EB, HP), jnp.float32),
        pltpu.VMEM((16,), jnp.float32),
        pltpu.VMEM((16,), jnp.float32),
        pltpu.VMEM_SHARED((NP, HP), jnp.float32),
        pltpu.SemaphoreType.DMA, pltpu.SemaphoreType.DMA,
    ],
    compiler_params=_sc_params,
)
def _k2(dst_hbm, src_hbm, al_hbm, ar_hbm, cal_hbm, car_hbm, z16_hbm,
        ex_hbm, dpart_hbm,
        dsti0, dsti1, srci0, srci1, gal0, gal1, gar0, gar1,
        calv, carv, dshared, sem0, sem1):
    cid = lax.axis_index("c")
    sid = lax.axis_index("s")
    wid = sid * NC + cid
    dsti, srci, gal, gar = (dsti0, dsti1), (srci0, srci1), (gal0, gal1), (gar0, gar1)
    sem = (sem0, sem1)
    pltpu.sync_copy(z16_hbm.at[pl.ds(sid * RPS, RPS)],
                    dshared.at[pl.ds(sid * RPS, RPS)])
    pltpu.sync_copy(cal_hbm.at[0], calv)
    pltpu.sync_copy(car_hbm.at[0], carv)
    plsc.subcore_barrier()
    c = jnp.maximum(calv[...] + carv[...], 0.0)
    e_base = wid * EB
    stride = NW * EB

    def issue(e0, b):
        pltpu.sync_copy(dst_hbm.at[pl.ds(e0, EB)], dsti[b])
        pltpu.sync_copy(src_hbm.at[pl.ds(e0, EB)], srci[b])
        pltpu.async_copy(al_hbm.at[dsti[b]], gal[b], sem[b])
        pltpu.async_copy(ar_hbm.at[srci[b]], gar[b], sem[b])

    def wait(b):
        pltpu.make_async_copy(al_hbm.at[dsti[b]], gal[b], sem[b]).wait()
        pltpu.make_async_copy(ar_hbm.at[srci[b]], gar[b], sem[b]).wait()

    @pl.when(e_base < E)
    def _():
        issue(e_base, 0)

    @pl.loop(0, _GMAX, step=2)
    def _(g):
        for b in (0, 1):
            e0 = e_base + (g + b) * stride

            @pl.when(e0 < E)
            def _():
                wait(b)

                @pl.when(e0 + stride < E)
                def _():
                    issue(e0 + stride, 1 - b)

                galb, garb = gal[b], gar[b]

                @pl.loop(0, EB, unroll=4)
                def _(j):
                    s = galb[j, :] + garb[j, :]
                    s = jnp.maximum(s, 0.2 * s)
                    galb[j, :] = jnp.exp(s - c)

                pltpu.sync_copy(galb, ex_hbm.at[pl.ds(e0, EB)])
                pltpu.sync_copy(galb, dshared.at[dsti[b]], add=True)

    plsc.subcore_barrier()
    pltpu.sync_copy(dshared.at[pl.ds(sid * RPS, RPS)],
                    dpart_hbm.at[cid, pl.ds(sid * RPS, RPS)])


def _k3_body(dp_ref, rcp_ref):
    d = dp_ref[0] + dp_ref[1]
    lanes = lax.broadcasted_iota(jnp.int32, (1, HP), 1)
    rcp_ref[...] = jnp.where(lanes < H, 1.0 / (d + 1e-16), 0.0)


_k3 = pl.pallas_call(
    _k3_body,
    out_shape=jax.ShapeDtypeStruct((NP, HP), jnp.float32),
)


@functools.partial(
    pl.kernel,
    out_type=jax.ShapeDtypeStruct((NC, NP, D), jnp.float32),
    mesh=_mesh,
    scratch_types=[
        pltpu.VMEM((EB,), jnp.int32), pltpu.VMEM((EB,), jnp.int32),
        pltpu.VMEM((EB,), jnp.int32), pltpu.VMEM((EB,), jnp.int32),
        pltpu.VMEM((EB, HP), jnp.float32), pltpu.VMEM((EB, HP), jnp.float32),
        pltpu.VMEM((EB, HP), jnp.float32), pltpu.VMEM((EB, HP), jnp.float32),
        pltpu.VMEM((EB, D), jnp.float32), pltpu.VMEM((EB, D), jnp.float32),
        pltpu.VMEM((EB,), jnp.float32),
        pltpu.VMEM_SHARED((NP, D), jnp.float32),
        pltpu.SemaphoreType.DMA, pltpu.SemaphoreType.DMA,
    ],
    compiler_params=_sc_params,
)
def _k4(dst_hbm, src_hbm, xl_hbm, ex_hbm, rcp_hbm, znd_hbm,
        opart_hbm,
        dsti0, dsti1, srci0, srci1, exb0, exb1, rcpb0, rcpb1,
        rows0, rows1, wbuf, oshared, sem0, sem1):
    cid = lax.axis_index("c")
    sid = lax.axis_index("s")
    wid = sid * NC + cid
    dsti, srci = (dsti0, dsti1), (srci0, srci1)
    exb, rcpb, rows = (exb0, exb1), (rcpb0, rcpb1), (rows0, rows1)
    sem = (sem0, sem1)
    pltpu.sync_copy(znd_hbm.at[pl.ds(sid * RPS, RPS)],
                    oshared.at[pl.ds(sid * RPS, RPS)])
    plsc.subcore_barrier()
    e_base = wid * EB
    stride = NW * EB
    lane = lax.iota(jnp.int32, 16)
    colh = [jnp.full((16,), h, jnp.int32) for h in range(H)]

    def issue(e0, b):
        pltpu.sync_copy(dst_hbm.at[pl.ds(e0, EB)], dsti[b])
        pltpu.sync_copy(src_hbm.at[pl.ds(e0, EB)], srci[b])
        pltpu.async_copy(ex_hbm.at[pl.ds(e0, EB)], exb[b], sem[b])
        pltpu.async_copy(rcp_hbm.at[dsti[b]], rcpb[b], sem[b])
        pltpu.async_copy(xl_hbm.at[srci[b]], rows[b], sem[b])

    def wait(e0, b):
        pltpu.make_async_copy(ex_hbm.at[pl.ds(e0, EB)], exb[b], sem[b]).wait()
        pltpu.make_async_copy(rcp_hbm.at[dsti[b]], rcpb[b], sem[b]).wait()
        pltpu.make_async_copy(xl_hbm.at[srci[b]], rows[b], sem[b]).wait()

    @pl.when(e_base < E)
    def _():
        issue(e_base, 0)

    @pl.loop(0, _GMAX, step=2)
    def _(g):
        for b in (0, 1):
            e0 = e_base + (g + b) * stride

            @pl.when(e0 < E)
            def _():
                wait(e0, b)

                @pl.when(e0 + stride < E)
                def _():
                    issue(e0 + stride, 1 - b)

                exbb, rcpbb, rowsb = exb[b], rcpb[b], rows[b]

                @pl.loop(0, EB, step=16)
                def _(j):
                    rowi = lane + j
                    acc = (plsc.load_gather(exbb, [rowi, colh[0]])
                           * plsc.load_gather(rcpbb, [rowi, colh[0]]))
                    for h in range(1, H):
                        acc = acc + (plsc.load_gather(exbb, [rowi, colh[h]])
                                     * plsc.load_gather(rcpbb, [rowi, colh[h]]))
                    wbuf[pl.ds(j, 16)] = 0.25 * acc

                @pl.loop(0, EB, unroll=2)
                def _(j):
                    wv = plsc.load_gather(wbuf, [jnp.full((16,), 0, jnp.int32) + j])
                    for k in range(D // 16):
                        rowsb[j, pl.ds(k * 16, 16)] = rowsb[j, pl.ds(k * 16, 16)] * wv

                pltpu.sync_copy(rowsb, oshared.at[dsti[b]], add=True)

    plsc.subcore_barrier()
    pltpu.sync_copy(oshared.at[pl.ds(sid * RPS, RPS)],
                    opart_hbm.at[cid, pl.ds(sid * RPS, RPS)])


def _k5_body(x_ref, o0_ref, o1_ref, b_ref, out_ref):
    out_ref[...] = x_ref[...] + o0_ref[0] + o1_ref[0] + b_ref[...]


_k5 = pl.pallas_call(
    _k5_body,
    grid=(_G1,),
    in_specs=[
        pl.BlockSpec((_R1, D), lambda i: (i, 0)),
        pl.BlockSpec((1, _R1, D), lambda i: (0, i, 0)),
        pl.BlockSpec((1, _R1, D), lambda i: (1, i, 0)),
        pl.BlockSpec((1, D), lambda i: (0, 0)),
    ],
    out_specs=pl.BlockSpec((_R1, D), lambda i: (i, 0)),
    out_shape=jax.ShapeDtypeStruct((N, D), jnp.float32),
)


def kernel(x, adj, W, att_l, att_r, bias, gamma, beta):
    Wt = W.T
    attl16 = jnp.pad(att_l, ((0, 0), (0, HP - H)))
    attr16 = jnp.pad(att_r, ((0, 0), (0, HP - H)))
    xl, al16, ar16, cal, car = _k1(x, Wt, attl16, attr16,
                                   gamma.reshape(1, D), beta.reshape(1, D))
    dst = adj[1]
    src = adj[0]
    z16 = jnp.zeros((NP, HP), jnp.float32)
    ex, dpart = _k2(dst, src, al16, ar16, cal, car, z16)
    rcp = _k3(dpart)
    znd = jnp.zeros((NP, D), jnp.float32)
    opart = _k4(dst, src, xl, ex, rcp, znd)
    return _k5(x, opart, opart, bias.reshape(1, D))
